# Initial kernel scaffold; baseline (speedup 1.0000x reference)
import jax
import jax.numpy as jnp
from jax.experimental import pallas as pl


def kernel(x, edge_index, edge_label_index, W1l, W1r, b1, W2l, W2r, b2):
    raise NotImplementedError("write your pallas kernel here")


# speedup vs baseline: 2.4847x; 2.4847x over previous
import functools

import jax
import jax.numpy as jnp
from jax import lax
from jax.experimental import pallas as pl
from jax.experimental.pallas import tpu as pltpu
from jax.experimental.pallas import tpu_sc as plsc

NC = 2
NS = 16
NW = NC * NS
LANES = 16
CH = 128
EPC = CH // 8
BS = 32


def _mesh():
    return plsc.VectorSubcoreMesh(
        core_axis_name="c", subcore_axis_name="s", num_cores=NC, num_subcores=NS
    )


def _sc_aggregate(table2, src2_r, dst_r, npad, with_count):
    dh = table2.shape[1]
    nch = src2_r.shape[2]
    dl = dh // LANES
    rows_pt = npad // NS
    nz = rows_pt // CH

    out_types = [jax.ShapeDtypeStruct((NC, npad, dh), jnp.float32)]
    if with_count:
        out_types.append(jax.ShapeDtypeStruct((npad, LANES), jnp.float32))
    scratch = [
        pltpu.VMEM((nch, CH), jnp.int32),
        pltpu.VMEM((nch, CH), jnp.int32),
        pltpu.VMEM((CH, dh), jnp.float32),
        pltpu.VMEM((CH, LANES), jnp.float32),
        pltpu.VMEM_SHARED((npad, dh), jnp.float32),
        pltpu.VMEM_SHARED((npad, LANES), jnp.float32),
        pltpu.SemaphoreType.DMA,
    ]

    @functools.partial(
        pl.kernel,
        out_type=tuple(out_types),
        mesh=_mesh(),
        scratch_types=scratch,
        compiler_params=pltpu.CompilerParams(use_tc_tiling_on_sc=False),
    )
    def run(table_h, src_h, dst_h, *rest):
        if with_count:
            agg_o, cnt_o, idx_s, idx_d, buf, ones, acc_sh, cnt_sh, sem = rest
        else:
            agg_o, idx_s, idx_d, buf, ones, acc_sh, cnt_sh, sem = rest
        c = lax.axis_index("c")
        s = lax.axis_index("s")

        zero = jnp.zeros((LANES,), jnp.float32)

        def zrow(i, _):
            for j in range(dl):
                buf[i, pl.ds(j * LANES, LANES)] = zero
            ones[i, pl.ds(0, LANES)] = zero
            return 0

        lax.fori_loop(0, CH, zrow, 0)
        base = s * rows_pt
        for k in range(nz):
            pltpu.sync_copy(buf, acc_sh.at[pl.ds(base + k * CH, CH)])
        if with_count:
            @pl.when(c == 0)
            def _():
                for k in range(nz):
                    pltpu.sync_copy(ones, cnt_sh.at[pl.ds(base + k * CH, CH)])

            one = jnp.full((LANES,), 1.0, jnp.float32)

            def orow(i, _):
                ones[i, pl.ds(0, LANES)] = one
                return 0

            lax.fori_loop(0, CH, orow, 0)

        pltpu.sync_copy(src_h.at[c, s], idx_s)
        pltpu.sync_copy(dst_h.at[s], idx_d)
        plsc.subcore_barrier()

        if with_count:
            def chunk(j, _):
                pltpu.async_copy(table_h.at[idx_s.at[j]], buf, sem).wait()
                pltpu.sync_copy(buf, acc_sh.at[idx_d.at[j]], add=True)

                @pl.when(c == 0)
                def _():
                    pltpu.sync_copy(ones, cnt_sh.at[idx_d.at[j]], add=True)
                return 0
        else:
            def chunk(j, _):
                pltpu.async_copy(table_h.at[idx_s.at[j]], buf, sem).wait()
                pltpu.sync_copy(buf, acc_sh.at[idx_d.at[j]], add=True)
                return 0

        lax.fori_loop(0, nch, chunk, 0)
        plsc.subcore_barrier()
        pltpu.sync_copy(
            acc_sh.at[pl.ds(base, rows_pt)], agg_o.at[c, pl.ds(base, rows_pt)]
        )
        if with_count:
            @pl.when(c == 0)
            def _():
                pltpu.sync_copy(
                    cnt_sh.at[pl.ds(base, rows_pt)], cnt_o.at[pl.ds(base, rows_pt)]
                )

    res = run(table2, src2_r, dst_r)
    if with_count:
        return res[0], res[1]
    return res[0] if isinstance(res, (tuple, list)) else res


def _sc_decode(z16, ia8, ib8):
    nb = ia8.shape[1]
    ng = nb * BS

    @functools.partial(
        pl.kernel,
        out_type=jax.ShapeDtypeStruct((NW, ng, LANES), jnp.float32),
        mesh=_mesh(),
        scratch_types=[
            pltpu.VMEM((BS, CH), jnp.int32),
            pltpu.VMEM((BS, CH), jnp.int32),
            pltpu.VMEM((CH, LANES), jnp.float32),
            pltpu.VMEM((CH, LANES), jnp.float32),
            pltpu.VMEM((ng, LANES), jnp.float32),
            pltpu.SemaphoreType.DMA,
            pltpu.SemaphoreType.DMA,
        ],
        compiler_params=pltpu.CompilerParams(
            use_tc_tiling_on_sc=False, needs_layout_passes=False
        ),
    )
    def run(z_h, ia_h, ib_h, out_h, idx_a, idx_b, bufa, bufb, obuf, sema, semb):
        c = lax.axis_index("c")
        s = lax.axis_index("s")
        wid = c * NS + s
        lanes = lax.iota(jnp.int32, LANES)

        def batch(bi, _):
            pltpu.sync_copy(ia_h.at[wid, bi], idx_a)
            pltpu.sync_copy(ib_h.at[wid, bi], idx_b)

            def chunk(j, _):
                ca = pltpu.async_copy(z_h.at[idx_a.at[j]], bufa, sema)
                cb = pltpu.async_copy(z_h.at[idx_b.at[j]], bufb, semb)
                ca.wait()
                cb.wait()
                vec = jnp.zeros((LANES,), jnp.float32)
                for ee in range(EPC):
                    acc = bufa[ee * 8] * bufb[ee * 8]
                    for k in range(1, 8):
                        acc = acc + bufa[ee * 8 + k] * bufb[ee * 8 + k]
                    vec = jnp.where(lanes == ee, jnp.sum(acc), vec)
                obuf[bi * BS + j] = vec
                return 0

            lax.fori_loop(0, BS, chunk, 0)
            return 0

        lax.fori_loop(0, nb, batch, 0)
        pltpu.sync_copy(obuf, out_h.at[wid])

    return run(z16, ia8, ib8)


def _tc_layer(agg, cnt, xin, wl, wr, b, relu, split_out):
    npad = agg.shape[1]
    dh = agg.shape[2]
    d = 2 * dh
    br = 1024
    grid = (npad // br,)
    b2 = b.reshape(1, d)

    def body(a_r, cnt_r, x_r, wl_r, wr_r, b_r, out_r):
        aggf = jnp.concatenate([a_r[0], a_r[1]], axis=1)
        xf = jnp.concatenate([x_r[0], x_r[1]], axis=1)
        cntc = jnp.clip(cnt_r[:, 0:1], 1.0, None)
        r = (
            jnp.dot(aggf / cntc, wl_r[...], preferred_element_type=jnp.float32)
            + jnp.dot(xf, wr_r[...], preferred_element_type=jnp.float32)
            + b_r[...]
        )
        if relu:
            r = jnp.maximum(r, 0.0)
        if split_out:
            out_r[0] = r[:, :dh]
            out_r[1] = r[:, dh:]
        else:
            out_r[...] = r

    if split_out:
        out_spec = pl.BlockSpec((NC, br, dh), lambda i: (0, i, 0))
        out_shape = jax.ShapeDtypeStruct((NC, npad, dh), jnp.float32)
    else:
        out_spec = pl.BlockSpec((br, d), lambda i: (i, 0))
        out_shape = jax.ShapeDtypeStruct((npad, d), jnp.float32)

    return pl.pallas_call(
        body,
        grid=grid,
        in_specs=[
            pl.BlockSpec((NC, br, dh), lambda i: (0, i, 0)),
            pl.BlockSpec((br, LANES), lambda i: (i, 0)),
            pl.BlockSpec((NC, br, dh), lambda i: (0, i, 0)),
            pl.BlockSpec((d, d), lambda i: (0, 0)),
            pl.BlockSpec((d, d), lambda i: (0, 0)),
            pl.BlockSpec((1, d), lambda i: (0, 0)),
        ],
        out_specs=out_spec,
        out_shape=out_shape,
    )(agg, cnt, xin, wl, wr, b2)


def kernel(x, edge_index, edge_label_index, W1l, W1r, b1, W2l, W2r, b2):
    n, d = x.shape
    e = edge_index.shape[1]
    dh = d // 2

    npad = -(-n // 256) * 256
    if npad == n:
        npad += 256
    epa = -(-e // (NS * CH)) * (NS * CH)
    nca = epa // (NS * CH)
    epd = -(-e // (NW * EPC * BS)) * (NW * EPC * BS)
    nb = epd // (NW * EPC * BS)

    src = jnp.pad(edge_index[0], (0, epa - e)).reshape(NS, nca, CH)
    src2 = jnp.stack([src, src + npad])
    dst = jnp.pad(edge_index[1], (0, epa - e), constant_values=n).reshape(NS, nca, CH)

    la = jnp.pad(edge_label_index[0], (0, epd - e)).reshape(NW, nb, BS, EPC)
    lb = jnp.pad(edge_label_index[1], (0, epd - e)).reshape(NW, nb, BS, EPC)
    k8 = jnp.arange(8, dtype=jnp.int32)
    ia8 = (la[..., None] * 8 + k8).reshape(NW, nb, BS, CH)
    ib8 = (lb[..., None] * 8 + k8).reshape(NW, nb, BS, CH)

    xp = jnp.pad(x, ((0, npad - n), (0, 0)))
    xsplit = jnp.stack([xp[:, :dh], xp[:, dh:]])

    agg1, cnt = _sc_aggregate(
        xsplit.reshape(2 * npad, dh), src2, dst, npad, with_count=True
    )
    hsplit = _tc_layer(agg1, cnt, xsplit, W1l, W1r, b1, relu=True, split_out=True)
    agg2 = _sc_aggregate(
        hsplit.reshape(2 * npad, dh), src2, dst, npad, with_count=False
    )
    z = _tc_layer(agg2, cnt, hsplit, W2l, W2r, b2, relu=False, split_out=False)
    out = _sc_decode(z.reshape(npad * 8, LANES), ia8, ib8)
    return out.reshape(-1)[:e]


# speedup vs baseline: 3.1359x; 1.2621x over previous
import functools

import jax
import jax.numpy as jnp
from jax import lax
from jax.experimental import pallas as pl
from jax.experimental.pallas import tpu as pltpu
from jax.experimental.pallas import tpu_sc as plsc

NC = 2
NS = 16
NW = NC * NS
LANES = 16
CH = 128
EPC = CH // 8
BS = 64
ANB = 3
DNB = 4


def _mesh():
    return plsc.VectorSubcoreMesh(
        core_axis_name="c", subcore_axis_name="s", num_cores=NC, num_subcores=NS
    )


def _sc_aggregate(table2, src2_r, dst_r, npad, with_count):
    dh = table2.shape[1]
    nch = src2_r.shape[2]
    dl = dh // LANES
    rows_pt = npad // NS
    nz = rows_pt // CH

    out_types = [jax.ShapeDtypeStruct((NC, npad, dh), jnp.float32)]
    if with_count:
        out_types.append(jax.ShapeDtypeStruct((npad, LANES), jnp.float32))
    scratch = [
        pltpu.VMEM((nch, CH), jnp.int32),
        pltpu.VMEM((nch, CH), jnp.int32),
        pltpu.VMEM((CH, LANES), jnp.float32),
        pltpu.VMEM_SHARED((npad, dh), jnp.float32),
        pltpu.VMEM_SHARED((npad, LANES), jnp.float32),
    ]
    for _ in range(ANB):
        scratch.append(pltpu.VMEM((CH, dh), jnp.float32))
    for _ in range(ANB):
        scratch.append(pltpu.SemaphoreType.DMA)

    @functools.partial(
        pl.kernel,
        out_type=tuple(out_types),
        mesh=_mesh(),
        scratch_types=scratch,
        compiler_params=pltpu.CompilerParams(use_tc_tiling_on_sc=False),
    )
    def run(table_h, src_h, dst_h, *rest):
        if with_count:
            agg_o, cnt_o, idx_s, idx_d, ones, acc_sh, cnt_sh = rest[:7]
            rest = rest[7:]
        else:
            agg_o, idx_s, idx_d, ones, acc_sh, cnt_sh = rest[:6]
            rest = rest[6:]
        bufs = rest[:ANB]
        sems = rest[ANB:2 * ANB]
        c = lax.axis_index("c")
        s = lax.axis_index("s")

        zero = jnp.zeros((LANES,), jnp.float32)
        buf0 = bufs[0]

        def zrow(i, _):
            for j in range(dl):
                buf0[i, pl.ds(j * LANES, LANES)] = zero
            ones[i, pl.ds(0, LANES)] = zero
            return 0

        lax.fori_loop(0, CH, zrow, 0)
        base = s * rows_pt
        for k in range(nz):
            pltpu.sync_copy(buf0, acc_sh.at[pl.ds(base + k * CH, CH)])
        if with_count:
            @pl.when(c == 0)
            def _():
                for k in range(nz):
                    pltpu.sync_copy(ones, cnt_sh.at[pl.ds(base + k * CH, CH)])

            one = jnp.full((LANES,), 1.0, jnp.float32)

            def orow(i, _):
                ones[i, pl.ds(0, LANES)] = one
                return 0

            lax.fori_loop(0, CH, orow, 0)

        pltpu.sync_copy(src_h.at[c, s], idx_s)
        pltpu.sync_copy(dst_h.at[s], idx_d)
        plsc.subcore_barrier()

        for b in range(ANB):
            pltpu.async_copy(table_h.at[idx_s.at[b]], bufs[b], sems[b])

        def group(g, _):
            for b in range(ANB):
                j = g * ANB + b
                pltpu.make_async_copy(
                    table_h.at[idx_s.at[j]], bufs[b], sems[b]
                ).wait()
                pltpu.sync_copy(bufs[b], acc_sh.at[idx_d.at[j]], add=True)
                if with_count:
                    @pl.when(c == 0)
                    def _():
                        pltpu.sync_copy(ones, cnt_sh.at[idx_d.at[j]], add=True)

                @pl.when(j + ANB < nch)
                def _():
                    pltpu.async_copy(table_h.at[idx_s.at[j + ANB]], bufs[b], sems[b])
            return 0

        lax.fori_loop(0, nch // ANB, group, 0)
        plsc.subcore_barrier()
        pltpu.sync_copy(
            acc_sh.at[pl.ds(base, rows_pt)], agg_o.at[c, pl.ds(base, rows_pt)]
        )
        if with_count:
            @pl.when(c == 0)
            def _():
                pltpu.sync_copy(
                    cnt_sh.at[pl.ds(base, rows_pt)], cnt_o.at[pl.ds(base, rows_pt)]
                )

    res = run(table2, src2_r, dst_r)
    if with_count:
        return res[0], res[1]
    return res[0] if isinstance(res, (tuple, list)) else res


def _sc_decode(z16, ia8, ib8):
    nb = ia8.shape[1]
    ng = nb * BS

    scratch = [
        pltpu.VMEM((2, BS, CH), jnp.int32),
        pltpu.VMEM((2, BS, CH), jnp.int32),
        pltpu.VMEM((ng, LANES), jnp.float32),
        pltpu.SemaphoreType.DMA,
        pltpu.SemaphoreType.DMA,
    ]
    for _ in range(DNB):
        scratch.append(pltpu.VMEM((CH, LANES), jnp.float32))
    for _ in range(DNB):
        scratch.append(pltpu.VMEM((CH, LANES), jnp.float32))
    for _ in range(2 * DNB):
        scratch.append(pltpu.SemaphoreType.DMA)

    @functools.partial(
        pl.kernel,
        out_type=jax.ShapeDtypeStruct((NW, ng, LANES), jnp.float32),
        mesh=_mesh(),
        scratch_types=scratch,
        compiler_params=pltpu.CompilerParams(
            use_tc_tiling_on_sc=False, needs_layout_passes=False
        ),
    )
    def run(z_h, ia_h, ib_h, out_h, idx_a, idx_b, obuf, isema, isemb, *rest):
        bufa = rest[:DNB]
        bufb = rest[DNB:2 * DNB]
        sema = rest[2 * DNB:3 * DNB]
        semb = rest[3 * DNB:4 * DNB]
        c = lax.axis_index("c")
        s = lax.axis_index("s")
        wid = c * NS + s
        lanes = lax.iota(jnp.int32, LANES)

        pltpu.sync_copy(ia_h.at[wid, 0], idx_a.at[0])
        pltpu.sync_copy(ib_h.at[wid, 0], idx_b.at[0])

        def batch(bi, _):
            bp = lax.rem(bi, 2)
            ia = idx_a.at[bp]
            ib = idx_b.at[bp]

            @pl.when(bi + 1 < nb)
            def _():
                bq = lax.rem(bi + 1, 2)
                pltpu.async_copy(ia_h.at[wid, bi + 1], idx_a.at[bq], isema)
                pltpu.async_copy(ib_h.at[wid, bi + 1], idx_b.at[bq], isemb)

            for b in range(DNB):
                pltpu.async_copy(z_h.at[ia.at[b]], bufa[b], sema[b])
                pltpu.async_copy(z_h.at[ib.at[b]], bufb[b], semb[b])

            def group(g, _):
                for b in range(DNB):
                    j = g * DNB + b
                    pltpu.make_async_copy(z_h.at[ia.at[j]], bufa[b], sema[b]).wait()
                    pltpu.make_async_copy(z_h.at[ib.at[j]], bufb[b], semb[b]).wait()
                    vec = jnp.zeros((LANES,), jnp.float32)
                    for ee in range(EPC):
                        acc = bufa[b][ee * 8] * bufb[b][ee * 8]
                        for k in range(1, 8):
                            acc = acc + bufa[b][ee * 8 + k] * bufb[b][ee * 8 + k]
                        vec = jnp.where(lanes == ee, jnp.sum(acc), vec)
                    obuf[bi * BS + j] = vec

                    @pl.when(j + DNB < BS)
                    def _():
                        pltpu.async_copy(z_h.at[ia.at[j + DNB]], bufa[b], sema[b])
                        pltpu.async_copy(z_h.at[ib.at[j + DNB]], bufb[b], semb[b])
                return 0

            lax.fori_loop(0, BS // DNB, group, 0)

            @pl.when(bi + 1 < nb)
            def _():
                bq = lax.rem(bi + 1, 2)
                pltpu.make_async_copy(ia_h.at[wid, bi + 1], idx_a.at[bq], isema).wait()
                pltpu.make_async_copy(ib_h.at[wid, bi + 1], idx_b.at[bq], isemb).wait()
            return 0

        lax.fori_loop(0, nb, batch, 0)
        pltpu.sync_copy(obuf, out_h.at[wid])

    return run(z16, ia8, ib8)


def _tc_layer(agg, cnt, xin, wl, wr, b, relu, split_out):
    npad = agg.shape[1]
    dh = agg.shape[2]
    d = 2 * dh
    br = 1024
    grid = (npad // br,)
    b2 = b.reshape(1, d)

    def body(a_r, cnt_r, x_r, wl_r, wr_r, b_r, out_r):
        aggf = jnp.concatenate([a_r[0], a_r[1]], axis=1)
        xf = jnp.concatenate([x_r[0], x_r[1]], axis=1)
        cntc = jnp.clip(cnt_r[:, 0:1], 1.0, None)
        r = (
            jnp.dot(aggf / cntc, wl_r[...], preferred_element_type=jnp.float32)
            + jnp.dot(xf, wr_r[...], preferred_element_type=jnp.float32)
            + b_r[...]
        )
        if relu:
            r = jnp.maximum(r, 0.0)
        if split_out:
            out_r[0] = r[:, :dh]
            out_r[1] = r[:, dh:]
        else:
            out_r[...] = r

    if split_out:
        out_spec = pl.BlockSpec((NC, br, dh), lambda i: (0, i, 0))
        out_shape = jax.ShapeDtypeStruct((NC, npad, dh), jnp.float32)
    else:
        out_spec = pl.BlockSpec((br, d), lambda i: (i, 0))
        out_shape = jax.ShapeDtypeStruct((npad, d), jnp.float32)

    return pl.pallas_call(
        body,
        grid=grid,
        in_specs=[
            pl.BlockSpec((NC, br, dh), lambda i: (0, i, 0)),
            pl.BlockSpec((br, LANES), lambda i: (i, 0)),
            pl.BlockSpec((NC, br, dh), lambda i: (0, i, 0)),
            pl.BlockSpec((d, d), lambda i: (0, 0)),
            pl.BlockSpec((d, d), lambda i: (0, 0)),
            pl.BlockSpec((1, d), lambda i: (0, 0)),
        ],
        out_specs=out_spec,
        out_shape=out_shape,
    )(agg, cnt, xin, wl, wr, b2)


def kernel(x, edge_index, edge_label_index, W1l, W1r, b1, W2l, W2r, b2):
    n, d = x.shape
    e = edge_index.shape[1]
    dh = d // 2

    npad = -(-n // 256) * 256
    if npad == n:
        npad += 256
    epa = -(-e // (NS * CH * ANB)) * (NS * CH * ANB)
    nca = epa // (NS * CH)
    epd = -(-e // (NW * EPC * BS)) * (NW * EPC * BS)
    nb = epd // (NW * EPC * BS)

    src = jnp.pad(edge_index[0], (0, epa - e)).reshape(NS, nca, CH)
    src2 = jnp.stack([src, src + npad])
    dst = jnp.pad(edge_index[1], (0, epa - e), constant_values=n).reshape(NS, nca, CH)

    la = jnp.pad(edge_label_index[0], (0, epd - e)).reshape(NW, nb, BS, EPC)
    lb = jnp.pad(edge_label_index[1], (0, epd - e)).reshape(NW, nb, BS, EPC)
    k8 = jnp.arange(8, dtype=jnp.int32)
    ia8 = (la[..., None] * 8 + k8).reshape(NW, nb, BS, CH)
    ib8 = (lb[..., None] * 8 + k8).reshape(NW, nb, BS, CH)

    xp = jnp.pad(x, ((0, npad - n), (0, 0)))
    xsplit = jnp.stack([xp[:, :dh], xp[:, dh:]])

    agg1, cnt = _sc_aggregate(
        xsplit.reshape(2 * npad, dh), src2, dst, npad, with_count=True
    )
    hsplit = _tc_layer(agg1, cnt, xsplit, W1l, W1r, b1, relu=True, split_out=True)
    agg2 = _sc_aggregate(
        hsplit.reshape(2 * npad, dh), src2, dst, npad, with_count=False
    )
    z = _tc_layer(agg2, cnt, hsplit, W2l, W2r, b2, relu=False, split_out=False)
    out = _sc_decode(z.reshape(npad * 8, LANES), ia8, ib8)
    return out.reshape(-1)[:e]
